# baseline (speedup 1.0000x reference)
import jax
import jax.numpy as jnp
from jax.experimental import pallas as pl
from jax.experimental.pallas import tpu as pltpu

_TS = 512


def _round_up(n: int, m: int) -> int:
    return ((n + m - 1) // m) * m


def _make_body(bt: int, ts: int, D_in: int, H1: int, H2: int, ns: int):
    def _body(x_ref, m_ref, w_ref, b_ref, o_ref, h1_ref, len_ref):
        s = pl.program_id(1)

        @pl.when(s == 0)
        def _():
            o_ref[...] = jnp.zeros_like(o_ref)
            len_ref[...] = jnp.zeros_like(len_ref)

        @pl.when(s < ns)
        def _():
            xb = x_ref[...].astype(jnp.bfloat16).reshape(bt * ts, -1)
            z1 = jnp.dot(xb, w_ref[0, :D_in, :H1],
                         preferred_element_type=jnp.float32)
            h1_ref[jax.lax.rem(s, 2)] = jnp.tanh(
                (z1 + b_ref[0, :, :H1]).astype(jnp.bfloat16))

        @pl.when(s > 0)
        def _():
            hp = h1_ref[jax.lax.rem(s + 1, 2)][...]
            z2 = jnp.dot(hp, w_ref[1, :H1, :H2],
                         preferred_element_type=jnp.float32)
            h2 = jnp.tanh((z2 + b_ref[1, :, :H2]).astype(jnp.bfloat16))
            h2 = h2.astype(jnp.float32).reshape(bt, ts, H2)
            m = m_ref[...].astype(jnp.float32)
            o_ref[...] += jnp.sum(h2 * m[:, :, None], axis=1)
            len_ref[...] += jnp.sum(m, axis=1, keepdims=True)

        @pl.when(s == ns)
        def _():
            o_ref[...] = o_ref[...] / jnp.maximum(len_ref[...], 1.0)

    return _body


def kernel(x, mask, w0, w1, b0, b1):
    B, S, D_in = x.shape
    H1 = w0.shape[1]
    H2 = w1.shape[1]

    Din_p, H1_p, H2_p = (_round_up(d, 128) for d in (D_in, H1, H2))

    ws = jnp.zeros((2, max(Din_p, H1_p), H1_p), jnp.bfloat16)
    ws = ws.at[0, :D_in, :H1].set(w0.astype(jnp.bfloat16))
    ws = ws.at[1, :H1, :H2].set(w1.astype(jnp.bfloat16))
    bs = jnp.zeros((2, 1, H1_p), jnp.float32)
    bs = bs.at[0, :, :H1].set(b0.reshape(1, -1).astype(jnp.float32))
    bs = bs.at[1, :, :H2].set(b1.reshape(1, -1).astype(jnp.float32))

    bt = 8 if B % 8 == 0 else B
    nb = B // bt
    ts = min(_TS, _round_up(S, 8))
    Sp = _round_up(S, ts)
    ns = Sp // ts

    xp = x
    mp = mask.astype(jnp.float32)
    if Sp != S or Din_p != D_in:
        xp = jnp.zeros((B, Sp, Din_p), x.dtype).at[:, :S, :D_in].set(x)
        mp = jnp.zeros((B, Sp), jnp.float32).at[:, :S].set(mp)

    out = pl.pallas_call(
        _make_body(bt, ts, Din_p, H1_p, H2_p, ns),
        out_shape=jax.ShapeDtypeStruct((B, H2_p), jnp.float32),
        grid_spec=pltpu.PrefetchScalarGridSpec(
            num_scalar_prefetch=0,
            grid=(nb, ns + 1),
            in_specs=[
                pl.BlockSpec((bt, ts, Din_p),
                             lambda i, s: (i, jnp.minimum(s, ns - 1), 0)),
                pl.BlockSpec((bt, ts),
                             lambda i, s: (i, jnp.maximum(s - 1, 0))),
                pl.BlockSpec(ws.shape, lambda i, s: (0, 0, 0)),
                pl.BlockSpec(bs.shape, lambda i, s: (0, 0, 0)),
            ],
            out_specs=pl.BlockSpec((bt, H2_p), lambda i, s: (i, 0)),
            scratch_shapes=[
                pltpu.VMEM((2, bt * ts, H1_p), jnp.bfloat16),
                pltpu.VMEM((bt, 1), jnp.float32),
            ],
        ),
        compiler_params=pltpu.CompilerParams(
            dimension_semantics=("arbitrary", "arbitrary"),
            vmem_limit_bytes=56 << 20,
        ),
    )(xp, mp, ws, bs)
    return out[:, :H2].astype(x.dtype)
